# baseline (speedup 1.0000x reference)
import functools

import jax
import jax.numpy as jnp
from jax import lax
from jax.experimental import pallas as pl
from jax.experimental.pallas import tpu as pltpu
from jax.experimental.pallas import tpu_sc as plsc

NC = 2
NS = 16
L = 16


def _sc_mesh():
    return plsc.VectorSubcoreMesh(
        core_axis_name="c", subcore_axis_name="s", num_cores=NC, num_subcores=NS
    )


C = 80
NB = 2
NR = 4
ZR = 32


def _make_deg_kernel(EPW, NP, W):
    NCH = EPW // C
    RPT = NP // NS
    assert EPW % (C * NR) == 0 and NP % NS == 0 and RPT % ZR == 0 and W % L == 0

    @functools.partial(
        pl.kernel,
        out_type=jax.ShapeDtypeStruct((NC * NP, W), jnp.float32),
        mesh=_sc_mesh(),
        scratch_types=(
            [pltpu.VMEM((C,), jnp.int32)] * NR
            + [
                pltpu.VMEM((C, W), jnp.float32),
                pltpu.VMEM((ZR, W), jnp.float32),
                pltpu.VMEM_SHARED((NP, W), jnp.float32),
            ]
            + [pltpu.SemaphoreType.DMA] * NR
        ),
    )
    def deg_kernel(dst_hbm, out_hbm, *scr):
        dsti = scr[:NR]
        ones_v, zb_v, acc_sh = scr[NR:NR + 3]
        dsem = scr[NR + 3:]
        c = lax.axis_index("c")
        s = lax.axis_index("s")
        wid = c * NS + s
        base = wid * EPW
        one_row = jnp.where(lax.iota(jnp.int32, L) == 0, 1.0, 0.0)
        zero_row = jnp.zeros((L,), jnp.float32)

        for m in range(NR):
            pltpu.async_copy(dst_hbm.at[pl.ds(base + m * C, C)], dsti[m], dsem[m])

        @pl.loop(0, C)
        def _(i):
            ones_v[i, pl.ds(0, L)] = one_row

            @pl.loop(L, W, step=L)
            def _(j):
                ones_v[i, pl.ds(j, L)] = zero_row

        @pl.loop(0, ZR)
        def _(i):
            @pl.loop(0, W, step=L)
            def _(j):
                zb_v[i, pl.ds(j, L)] = zero_row

        @pl.loop(0, RPT, step=ZR)
        def _(r):
            pltpu.sync_copy(zb_v, acc_sh.at[pl.ds(s * RPT + r, ZR)])

        plsc.subcore_barrier()

        @pl.loop(0, NCH, step=NR)
        def _(g):
            for j in range(NR):
                k = g + j
                pltpu.make_async_copy(
                    dst_hbm.at[pl.ds(base + k * C, C)], dsti[j], dsem[j]
                ).wait()
                pltpu.sync_copy(ones_v, acc_sh.at[dsti[j]], add=True)
                nk = k + NR

                @pl.when(nk < NCH)
                def _():
                    pltpu.async_copy(
                        dst_hbm.at[pl.ds(base + nk * C, C)], dsti[j], dsem[j]
                    )

        plsc.subcore_barrier()
        pltpu.sync_copy(
            acc_sh.at[pl.ds(s * RPT, RPT)],
            out_hbm.at[pl.ds(c * NP + s * RPT, RPT)],
        )

    return deg_kernel


def _make_mp_kernel(EPW, D, NP):
    NCH = EPW // C
    RPT = NP // NS
    assert EPW % (C * NR) == 0 and D % L == 0
    assert NP % NS == 0 and RPT % ZR == 0

    @functools.partial(
        pl.kernel,
        out_type=jax.ShapeDtypeStruct((NC * NP, D), jnp.float32),
        mesh=_sc_mesh(),
        scratch_types=(
            [pltpu.VMEM((EPW,), jnp.int32)]
            + [pltpu.VMEM((C,), jnp.int32)] * NR
            + [pltpu.VMEM((C, D), jnp.float32)] * NB
            + [
                pltpu.VMEM((ZR, D), jnp.float32),
                pltpu.VMEM_SHARED((NP, D), jnp.float32),
            ]
            + [pltpu.SemaphoreType.DMA] * (NB + NR)
        ),
    )
    def mp_kernel(g_hbm, src_hbm, dst_hbm, out_hbm, *scr):
        srci = scr[0]
        dsti = scr[1:1 + NR]
        rows = scr[1 + NR:1 + NR + NB]
        zb_v, acc_sh = scr[1 + NR + NB:1 + NR + NB + 2]
        gsem = scr[1 + NR + NB + 2:1 + NR + NB + 2 + NB]
        dsem = scr[1 + NR + NB + 2 + NB:]
        c = lax.axis_index("c")
        s = lax.axis_index("s")
        wid = c * NS + s
        base = wid * EPW
        zero_row = jnp.zeros((L,), jnp.float32)

        for m in range(NR):
            pltpu.async_copy(dst_hbm.at[pl.ds(base + m * C, C)], dsti[m], dsem[m])
        pltpu.sync_copy(src_hbm.at[pl.ds(base, EPW)], srci)

        @pl.loop(0, ZR)
        def _(i):
            @pl.loop(0, D, step=L)
            def _(j):
                zb_v[i, pl.ds(j, L)] = zero_row

        @pl.loop(0, RPT, step=ZR)
        def _(r):
            pltpu.sync_copy(zb_v, acc_sh.at[pl.ds(s * RPT + r, ZR)])

        plsc.subcore_barrier()

        for b in range(NB):
            pltpu.async_copy(
                g_hbm.at[srci.at[pl.ds(b * C, C)]], rows[b], gsem[b]
            )

        @pl.loop(0, NCH, step=NR)
        def _(g):
            for j in range(NR):
                k = g + j
                b = j % NB
                pltpu.make_async_copy(
                    g_hbm.at[srci.at[pl.ds(k * C, C)]], rows[b], gsem[b]
                ).wait()
                pltpu.make_async_copy(
                    dst_hbm.at[pl.ds(base + k * C, C)], dsti[j], dsem[j]
                ).wait()
                pltpu.sync_copy(rows[b], acc_sh.at[dsti[j]], add=True)
                nkd = k + NR

                @pl.when(nkd < NCH)
                def _():
                    pltpu.async_copy(
                        dst_hbm.at[pl.ds(base + nkd * C, C)], dsti[j], dsem[j]
                    )

                nkg = k + NB

                @pl.when(nkg < NCH)
                def _():
                    pltpu.async_copy(
                        g_hbm.at[srci.at[pl.ds(nkg * C, C)]], rows[b], gsem[b]
                    )

        plsc.subcore_barrier()
        pltpu.sync_copy(
            acc_sh.at[pl.ds(s * RPT, RPT)],
            out_hbm.at[pl.ds(c * NP + s * RPT, RPT)],
        )

    return mp_kernel


def _mm_body(x_ref, w_ref, o_ref):
    o_ref[...] = jnp.dot(x_ref[...], w_ref[...],
                         preferred_element_type=jnp.float32)


def _scale_body(h_ref, dp_ref, g_ref, dinv_ref):
    p = dp_ref[...]
    deg = jnp.sum(p[0] + p[1], axis=1, keepdims=True) + 1.0
    dinv = lax.rsqrt(deg)
    dinv_ref[...] = dinv
    g_ref[...] = dinv * h_ref[...]


def _mid_body(a_ref, g_ref, dinv_ref, b_ref, w_ref, o_ref):
    dinv = dinv_ref[...]
    acc = a_ref[0] + a_ref[1] + g_ref[...]
    z = jnp.maximum(dinv * acc + b_ref[...], 0.0)
    o_ref[...] = dinv * jnp.dot(z, w_ref[...],
                                preferred_element_type=jnp.float32)


def _out_body(a_ref, g_ref, dinv_ref, b_ref, o_ref):
    o_ref[...] = dinv_ref[...] * (a_ref[0] + a_ref[1] + g_ref[...]) + b_ref[...]


def kernel(x, edge_index, W1, b1, W2, b2):
    n, K = x.shape
    E = edge_index.shape[1]
    RB = 1000
    assert n % RB == 0
    G = n // RB
    NP = -(-n // 2048) * 2048
    NW = NC * NS
    EPW = -(-E // (NW * C * NR)) * C * NR
    PAD = EPW * NW - E
    if PAD > 0 and NP == n:
        NP += 2048
    src = edge_index[0]
    dst = edge_index[1]
    if PAD > 0:
        src = jnp.concatenate([src, jnp.zeros((PAD,), jnp.int32)])
        dump = n + (jnp.arange(PAD, dtype=jnp.int32) % (NP - n))
        dst = jnp.concatenate([dst, dump])

    deg_parts = _make_deg_kernel(EPW, NP, K)(dst)

    h1 = pl.pallas_call(
        _mm_body,
        grid=(G,),
        in_specs=[pl.BlockSpec((RB, K), lambda i: (i, 0)),
                  pl.BlockSpec((K, K), lambda i: (0, 0))],
        out_specs=pl.BlockSpec((RB, K), lambda i: (i, 0)),
        out_shape=jax.ShapeDtypeStruct((n, K), jnp.float32),
    )(x, W1)

    g1, dinv = pl.pallas_call(
        _scale_body,
        grid=(G,),
        in_specs=[pl.BlockSpec((RB, K), lambda i: (i, 0)),
                  pl.BlockSpec((NC, RB, K), lambda i: (0, i, 0))],
        out_specs=[pl.BlockSpec((RB, K), lambda i: (i, 0)),
                   pl.BlockSpec((RB, 1), lambda i: (i, 0))],
        out_shape=[jax.ShapeDtypeStruct((n, K), jnp.float32),
                   jax.ShapeDtypeStruct((n, 1), jnp.float32)],
    )(h1, deg_parts.reshape(NC, NP, K)[:, :n, :])

    mp = _make_mp_kernel(EPW, K, NP)
    acc1 = mp(g1, src, dst).reshape(NC, NP, K)[:, :n, :]

    g2 = pl.pallas_call(
        _mid_body,
        grid=(G,),
        in_specs=[pl.BlockSpec((NC, RB, K), lambda i: (0, i, 0)),
                  pl.BlockSpec((RB, K), lambda i: (i, 0)),
                  pl.BlockSpec((RB, 1), lambda i: (i, 0)),
                  pl.BlockSpec((1, K), lambda i: (0, 0)),
                  pl.BlockSpec((K, K), lambda i: (0, 0))],
        out_specs=pl.BlockSpec((RB, K), lambda i: (i, 0)),
        out_shape=jax.ShapeDtypeStruct((n, K), jnp.float32),
    )(acc1, g1, dinv, b1.reshape(1, K), W2)

    acc2 = mp(g2, src, dst).reshape(NC, NP, K)[:, :n, :]

    out = pl.pallas_call(
        _out_body,
        grid=(G,),
        in_specs=[pl.BlockSpec((NC, RB, K), lambda i: (0, i, 0)),
                  pl.BlockSpec((RB, K), lambda i: (i, 0)),
                  pl.BlockSpec((RB, 1), lambda i: (i, 0)),
                  pl.BlockSpec((1, K), lambda i: (0, 0))],
        out_specs=pl.BlockSpec((RB, K), lambda i: (i, 0)),
        out_shape=jax.ShapeDtypeStruct((n, K), jnp.float32),
    )(acc2, g2, dinv, b2.reshape(1, K))

    return out


# speedup vs baseline: 1.0351x; 1.0351x over previous
import functools

import jax
import jax.numpy as jnp
from jax import lax
from jax.experimental import pallas as pl
from jax.experimental.pallas import tpu as pltpu
from jax.experimental.pallas import tpu_sc as plsc

NC = 2
NS = 16
L = 16


def _sc_mesh():
    return plsc.VectorSubcoreMesh(
        core_axis_name="c", subcore_axis_name="s", num_cores=NC, num_subcores=NS
    )


C = 80
NB = 2
NR = 4
ZR = 32


def _make_deg_kernel(EPW, NP, W):
    NCH = EPW // C
    RPT = NP // NS
    assert EPW % (C * NR) == 0 and NP % NS == 0 and RPT % ZR == 0 and W % L == 0

    @functools.partial(
        pl.kernel,
        out_type=jax.ShapeDtypeStruct((NC * NP, W), jnp.float32),
        mesh=_sc_mesh(),
        scratch_types=(
            [pltpu.VMEM((C,), jnp.int32)] * NR
            + [
                pltpu.VMEM((C, W), jnp.float32),
                pltpu.VMEM((ZR, W), jnp.float32),
                pltpu.VMEM_SHARED((NP, W), jnp.float32),
            ]
            + [pltpu.SemaphoreType.DMA] * NR
        ),
    )
    def deg_kernel(dst_hbm, out_hbm, *scr):
        dsti = scr[:NR]
        ones_v, zb_v, acc_sh = scr[NR:NR + 3]
        dsem = scr[NR + 3:]
        c = lax.axis_index("c")
        s = lax.axis_index("s")
        wid = c * NS + s
        base = wid * EPW
        one_row = jnp.where(lax.iota(jnp.int32, L) == 0, 1.0, 0.0)
        zero_row = jnp.zeros((L,), jnp.float32)

        for m in range(NR):
            pltpu.async_copy(dst_hbm.at[pl.ds(base + m * C, C)], dsti[m], dsem[m])

        @pl.loop(0, C)
        def _(i):
            ones_v[i, pl.ds(0, L)] = one_row

            @pl.loop(L, W, step=L)
            def _(j):
                ones_v[i, pl.ds(j, L)] = zero_row

        @pl.loop(0, ZR)
        def _(i):
            @pl.loop(0, W, step=L)
            def _(j):
                zb_v[i, pl.ds(j, L)] = zero_row

        @pl.loop(0, RPT, step=ZR)
        def _(r):
            pltpu.sync_copy(zb_v, acc_sh.at[pl.ds(s * RPT + r, ZR)])

        plsc.subcore_barrier()

        @pl.loop(0, NCH, step=NR)
        def _(g):
            for j in range(NR):
                k = g + j
                pltpu.make_async_copy(
                    dst_hbm.at[pl.ds(base + k * C, C)], dsti[j], dsem[j]
                ).wait()
                pltpu.sync_copy(ones_v, acc_sh.at[dsti[j]], add=True)
                nk = k + NR

                @pl.when(nk < NCH)
                def _():
                    pltpu.async_copy(
                        dst_hbm.at[pl.ds(base + nk * C, C)], dsti[j], dsem[j]
                    )

        plsc.subcore_barrier()
        pltpu.sync_copy(
            acc_sh.at[pl.ds(s * RPT, RPT)],
            out_hbm.at[pl.ds(c * NP + s * RPT, RPT)],
        )

    return deg_kernel


def _make_mp_kernel(EPW, D, NP):
    NCH = EPW // C
    RPT = NP // NS
    assert EPW % (C * NR) == 0 and D % L == 0
    assert NP % NS == 0 and RPT % ZR == 0

    @functools.partial(
        pl.kernel,
        out_type=jax.ShapeDtypeStruct((NC * NP, D), jnp.float32),
        mesh=_sc_mesh(),
        scratch_types=(
            [pltpu.VMEM((EPW,), jnp.int32)]
            + [pltpu.VMEM((C,), jnp.int32)] * NR
            + [pltpu.VMEM((C, D), jnp.float32)] * NB
            + [
                pltpu.VMEM((ZR, D), jnp.float32),
                pltpu.VMEM_SHARED((NP, D), jnp.float32),
            ]
            + [pltpu.SemaphoreType.DMA] * (NB + NR)
        ),
    )
    def mp_kernel(g_hbm, src_hbm, dst_hbm, out_hbm, *scr):
        srci = scr[0]
        dsti = scr[1:1 + NR]
        rows = scr[1 + NR:1 + NR + NB]
        zb_v, acc_sh = scr[1 + NR + NB:1 + NR + NB + 2]
        gsem = scr[1 + NR + NB + 2:1 + NR + NB + 2 + NB]
        dsem = scr[1 + NR + NB + 2 + NB:]
        c = lax.axis_index("c")
        s = lax.axis_index("s")
        wid = c * NS + s
        base = wid * EPW
        zero_row = jnp.zeros((L,), jnp.float32)

        for m in range(NR):
            pltpu.async_copy(dst_hbm.at[pl.ds(base + m * C, C)], dsti[m], dsem[m])
        pltpu.sync_copy(src_hbm.at[pl.ds(base, EPW)], srci)

        @pl.loop(0, ZR)
        def _(i):
            @pl.loop(0, D, step=L)
            def _(j):
                zb_v[i, pl.ds(j, L)] = zero_row

        @pl.loop(0, RPT, step=ZR)
        def _(r):
            pltpu.sync_copy(zb_v, acc_sh.at[pl.ds(s * RPT + r, ZR)])

        plsc.subcore_barrier()

        for b in range(NB):
            pltpu.async_copy(
                g_hbm.at[srci.at[pl.ds(b * C, C)]], rows[b], gsem[b]
            )

        @pl.loop(0, NCH, step=NR)
        def _(g):
            for j in range(NR):
                k = g + j
                b = j % NB
                pltpu.make_async_copy(
                    g_hbm.at[srci.at[pl.ds(k * C, C)]], rows[b], gsem[b]
                ).wait()
                pltpu.make_async_copy(
                    dst_hbm.at[pl.ds(base + k * C, C)], dsti[j], dsem[j]
                ).wait()
                pltpu.sync_copy(rows[b], acc_sh.at[dsti[j]], add=True)
                nkd = k + NR

                @pl.when(nkd < NCH)
                def _():
                    pltpu.async_copy(
                        dst_hbm.at[pl.ds(base + nkd * C, C)], dsti[j], dsem[j]
                    )

                nkg = k + NB

                @pl.when(nkg < NCH)
                def _():
                    pltpu.async_copy(
                        g_hbm.at[srci.at[pl.ds(nkg * C, C)]], rows[b], gsem[b]
                    )

        plsc.subcore_barrier()
        pltpu.sync_copy(
            acc_sh.at[pl.ds(s * RPT, RPT)],
            out_hbm.at[pl.ds(c * NP + s * RPT, RPT)],
        )

    return mp_kernel


def _mm_body(x_ref, w_ref, o_ref):
    o_ref[...] = jnp.dot(x_ref[...], w_ref[...],
                         preferred_element_type=jnp.float32)


def _scale_body(h_ref, dp_ref, g_ref, dinv_ref):
    p = dp_ref[...]
    deg = jnp.sum(p[0] + p[1], axis=1, keepdims=True) + 1.0
    dinv = lax.rsqrt(deg)
    dinv_ref[...] = dinv
    g_ref[...] = dinv * h_ref[...]


def _mid_body(a_ref, g_ref, dinv_ref, b_ref, w_ref, o_ref):
    dinv = dinv_ref[...]
    acc = a_ref[0] + a_ref[1] + g_ref[...]
    z = jnp.maximum(dinv * acc + b_ref[...], 0.0)
    o_ref[...] = dinv * jnp.dot(z, w_ref[...],
                                preferred_element_type=jnp.float32)


def _out_body(a_ref, g_ref, dinv_ref, b_ref, o_ref):
    o_ref[...] = dinv_ref[...] * (a_ref[0] + a_ref[1] + g_ref[...]) + b_ref[...]


def kernel(x, edge_index, W1, b1, W2, b2):
    n, K = x.shape
    E = edge_index.shape[1]
    RB = 1000
    assert n % RB == 0
    G = n // RB
    NP = -(-n // 2048) * 2048
    NW = NC * NS
    EPW = -(-E // (NW * C * NR)) * C * NR
    PAD = EPW * NW - E
    if PAD > 0 and NP == n:
        NP += 2048
    src = edge_index[0]
    dst = edge_index[1]
    if PAD > 0:
        ppw = PAD // NW
        dump = n + (jnp.arange(PAD, dtype=jnp.int32) % (NP - n))
        if E % NW == 0 and PAD % NW == 0:
            src = jnp.concatenate(
                [src.reshape(NW, -1), jnp.zeros((NW, ppw), jnp.int32)], axis=1
            ).reshape(-1)
            dst = jnp.concatenate(
                [dst.reshape(NW, -1), dump.reshape(NW, ppw)], axis=1
            ).reshape(-1)
        else:
            src = jnp.concatenate([src, jnp.zeros((PAD,), jnp.int32)])
            dst = jnp.concatenate([dst, dump])

    deg_parts = _make_deg_kernel(EPW, NP, K)(dst)

    h1 = pl.pallas_call(
        _mm_body,
        grid=(G,),
        in_specs=[pl.BlockSpec((RB, K), lambda i: (i, 0)),
                  pl.BlockSpec((K, K), lambda i: (0, 0))],
        out_specs=pl.BlockSpec((RB, K), lambda i: (i, 0)),
        out_shape=jax.ShapeDtypeStruct((n, K), jnp.float32),
    )(x, W1)

    g1, dinv = pl.pallas_call(
        _scale_body,
        grid=(G,),
        in_specs=[pl.BlockSpec((RB, K), lambda i: (i, 0)),
                  pl.BlockSpec((NC, RB, K), lambda i: (0, i, 0))],
        out_specs=[pl.BlockSpec((RB, K), lambda i: (i, 0)),
                   pl.BlockSpec((RB, 1), lambda i: (i, 0))],
        out_shape=[jax.ShapeDtypeStruct((n, K), jnp.float32),
                   jax.ShapeDtypeStruct((n, 1), jnp.float32)],
    )(h1, deg_parts.reshape(NC, NP, K)[:, :n, :])

    mp = _make_mp_kernel(EPW, K, NP)
    acc1 = mp(g1, src, dst).reshape(NC, NP, K)[:, :n, :]

    g2 = pl.pallas_call(
        _mid_body,
        grid=(G,),
        in_specs=[pl.BlockSpec((NC, RB, K), lambda i: (0, i, 0)),
                  pl.BlockSpec((RB, K), lambda i: (i, 0)),
                  pl.BlockSpec((RB, 1), lambda i: (i, 0)),
                  pl.BlockSpec((1, K), lambda i: (0, 0)),
                  pl.BlockSpec((K, K), lambda i: (0, 0))],
        out_specs=pl.BlockSpec((RB, K), lambda i: (i, 0)),
        out_shape=jax.ShapeDtypeStruct((n, K), jnp.float32),
    )(acc1, g1, dinv, b1.reshape(1, K), W2)

    acc2 = mp(g2, src, dst).reshape(NC, NP, K)[:, :n, :]

    out = pl.pallas_call(
        _out_body,
        grid=(G,),
        in_specs=[pl.BlockSpec((NC, RB, K), lambda i: (0, i, 0)),
                  pl.BlockSpec((RB, K), lambda i: (i, 0)),
                  pl.BlockSpec((RB, 1), lambda i: (i, 0)),
                  pl.BlockSpec((1, K), lambda i: (0, 0))],
        out_specs=pl.BlockSpec((RB, K), lambda i: (i, 0)),
        out_shape=jax.ShapeDtypeStruct((n, K), jnp.float32),
    )(acc2, g2, dinv, b2.reshape(1, K))

    return out


# speedup vs baseline: 1.2944x; 1.2505x over previous
import functools

import jax
import jax.numpy as jnp
from jax import lax
from jax.experimental import pallas as pl
from jax.experimental.pallas import tpu as pltpu
from jax.experimental.pallas import tpu_sc as plsc

NC = 2
NS = 16
L = 16


def _sc_mesh():
    return plsc.VectorSubcoreMesh(
        core_axis_name="c", subcore_axis_name="s", num_cores=NC, num_subcores=NS
    )


C = 48
NB = 4
NR = 4
ZR = 32


def _make_deg_kernel(EPW, NP, W):
    NCH = EPW // C
    RPT = NP // NS
    assert EPW % (C * NR) == 0 and NP % NS == 0 and RPT % ZR == 0 and W % L == 0

    @functools.partial(
        pl.kernel,
        out_type=jax.ShapeDtypeStruct((NC * NP, W), jnp.float32),
        mesh=_sc_mesh(),
        scratch_types=(
            [pltpu.VMEM((C,), jnp.int32)] * NR
            + [
                pltpu.VMEM((C, W), jnp.float32),
                pltpu.VMEM((ZR, W), jnp.float32),
                pltpu.VMEM_SHARED((NP, W), jnp.float32),
            ]
            + [pltpu.SemaphoreType.DMA] * NR
        ),
    )
    def deg_kernel(dst_hbm, out_hbm, *scr):
        dsti = scr[:NR]
        ones_v, zb_v, acc_sh = scr[NR:NR + 3]
        dsem = scr[NR + 3:]
        c = lax.axis_index("c")
        s = lax.axis_index("s")
        wid = c * NS + s
        base = wid * EPW
        one_row = jnp.where(lax.iota(jnp.int32, L) == 0, 1.0, 0.0)
        zero_row = jnp.zeros((L,), jnp.float32)

        for m in range(NR):
            pltpu.async_copy(dst_hbm.at[pl.ds(base + m * C, C)], dsti[m], dsem[m])

        @pl.loop(0, C)
        def _(i):
            ones_v[i, pl.ds(0, L)] = one_row

            @pl.loop(L, W, step=L)
            def _(j):
                ones_v[i, pl.ds(j, L)] = zero_row

        @pl.loop(0, ZR)
        def _(i):
            @pl.loop(0, W, step=L)
            def _(j):
                zb_v[i, pl.ds(j, L)] = zero_row

        @pl.loop(0, RPT, step=ZR)
        def _(r):
            pltpu.sync_copy(zb_v, acc_sh.at[pl.ds(s * RPT + r, ZR)])

        plsc.subcore_barrier()

        @pl.loop(0, NCH, step=NR)
        def _(g):
            for j in range(NR):
                k = g + j
                pltpu.make_async_copy(
                    dst_hbm.at[pl.ds(base + k * C, C)], dsti[j], dsem[j]
                ).wait()
                pltpu.sync_copy(ones_v, acc_sh.at[dsti[j]], add=True)
                nk = k + NR

                @pl.when(nk < NCH)
                def _():
                    pltpu.async_copy(
                        dst_hbm.at[pl.ds(base + nk * C, C)], dsti[j], dsem[j]
                    )

        plsc.subcore_barrier()
        pltpu.sync_copy(
            acc_sh.at[pl.ds(s * RPT, RPT)],
            out_hbm.at[pl.ds(c * NP + s * RPT, RPT)],
        )

    return deg_kernel


def _make_mp_kernel(EPW, D, NP):
    NCH = EPW // C
    RPT = NP // NS
    assert EPW % (C * NR) == 0 and D % L == 0
    assert NP % NS == 0 and RPT % ZR == 0

    @functools.partial(
        pl.kernel,
        out_type=jax.ShapeDtypeStruct((NC * NP, D), jnp.float32),
        mesh=_sc_mesh(),
        scratch_types=(
            [pltpu.VMEM((EPW,), jnp.int32)]
            + [pltpu.VMEM((C,), jnp.int32)] * NR
            + [pltpu.VMEM((C, D), jnp.float32)] * NB
            + [
                pltpu.VMEM((ZR, D), jnp.float32),
                pltpu.VMEM_SHARED((NP, D), jnp.float32),
            ]
            + [pltpu.SemaphoreType.DMA] * (NB + NR)
        ),
    )
    def mp_kernel(g_hbm, src_hbm, dst_hbm, out_hbm, *scr):
        srci = scr[0]
        dsti = scr[1:1 + NR]
        rows = scr[1 + NR:1 + NR + NB]
        zb_v, acc_sh = scr[1 + NR + NB:1 + NR + NB + 2]
        gsem = scr[1 + NR + NB + 2:1 + NR + NB + 2 + NB]
        dsem = scr[1 + NR + NB + 2 + NB:]
        c = lax.axis_index("c")
        s = lax.axis_index("s")
        wid = c * NS + s
        base = wid * EPW
        zero_row = jnp.zeros((L,), jnp.float32)

        for m in range(NR):
            pltpu.async_copy(dst_hbm.at[pl.ds(base + m * C, C)], dsti[m], dsem[m])
        pltpu.sync_copy(src_hbm.at[pl.ds(base, EPW)], srci)

        @pl.loop(0, ZR)
        def _(i):
            @pl.loop(0, D, step=L)
            def _(j):
                zb_v[i, pl.ds(j, L)] = zero_row

        @pl.loop(0, RPT, step=ZR)
        def _(r):
            pltpu.sync_copy(zb_v, acc_sh.at[pl.ds(s * RPT + r, ZR)])

        plsc.subcore_barrier()

        for b in range(NB):
            pltpu.async_copy(
                g_hbm.at[srci.at[pl.ds(b * C, C)]], rows[b], gsem[b]
            )

        @pl.loop(0, NCH, step=NR)
        def _(g):
            for j in range(NR):
                k = g + j
                b = j % NB
                pltpu.make_async_copy(
                    g_hbm.at[srci.at[pl.ds(k * C, C)]], rows[b], gsem[b]
                ).wait()
                pltpu.make_async_copy(
                    dst_hbm.at[pl.ds(base + k * C, C)], dsti[j], dsem[j]
                ).wait()
                pltpu.sync_copy(rows[b], acc_sh.at[dsti[j]], add=True)
                nkd = k + NR

                @pl.when(nkd < NCH)
                def _():
                    pltpu.async_copy(
                        dst_hbm.at[pl.ds(base + nkd * C, C)], dsti[j], dsem[j]
                    )

                nkg = k + NB

                @pl.when(nkg < NCH)
                def _():
                    pltpu.async_copy(
                        g_hbm.at[srci.at[pl.ds(nkg * C, C)]], rows[b], gsem[b]
                    )

        plsc.subcore_barrier()
        pltpu.sync_copy(
            acc_sh.at[pl.ds(s * RPT, RPT)],
            out_hbm.at[pl.ds(c * NP + s * RPT, RPT)],
        )

    return mp_kernel


def _mm_body(x_ref, w_ref, o_ref):
    o_ref[...] = jnp.dot(x_ref[...], w_ref[...],
                         preferred_element_type=jnp.float32)


def _scale_body(h_ref, dp_ref, g_ref, dinv_ref):
    p = dp_ref[...]
    deg = jnp.sum(p[0] + p[1], axis=1, keepdims=True) + 1.0
    dinv = lax.rsqrt(deg)
    dinv_ref[...] = dinv
    g_ref[...] = dinv * h_ref[...]


def _mid_body(a_ref, g_ref, dinv_ref, b_ref, w_ref, o_ref):
    dinv = dinv_ref[...]
    acc = a_ref[0] + a_ref[1] + g_ref[...]
    z = jnp.maximum(dinv * acc + b_ref[...], 0.0)
    o_ref[...] = dinv * jnp.dot(z, w_ref[...],
                                preferred_element_type=jnp.float32)


def _out_body(a_ref, g_ref, dinv_ref, b_ref, o_ref):
    o_ref[...] = dinv_ref[...] * (a_ref[0] + a_ref[1] + g_ref[...]) + b_ref[...]


def kernel(x, edge_index, W1, b1, W2, b2):
    n, K = x.shape
    E = edge_index.shape[1]
    RB = 1000
    assert n % RB == 0
    G = n // RB
    NP = -(-n // 2048) * 2048
    NW = NC * NS
    EPW = -(-E // (NW * C * NR)) * C * NR
    PAD = EPW * NW - E
    if PAD > 0 and NP == n:
        NP += 2048
    src = edge_index[0]
    dst = edge_index[1]
    if PAD > 0:
        ppw = PAD // NW
        dump = n + (jnp.arange(PAD, dtype=jnp.int32) % (NP - n))
        if E % NW == 0 and PAD % NW == 0:
            src = jnp.concatenate(
                [src.reshape(NW, -1), jnp.zeros((NW, ppw), jnp.int32)], axis=1
            ).reshape(-1)
            dst = jnp.concatenate(
                [dst.reshape(NW, -1), dump.reshape(NW, ppw)], axis=1
            ).reshape(-1)
        else:
            src = jnp.concatenate([src, jnp.zeros((PAD,), jnp.int32)])
            dst = jnp.concatenate([dst, dump])

    deg_parts = _make_deg_kernel(EPW, NP, K)(dst)

    h1 = pl.pallas_call(
        _mm_body,
        grid=(G,),
        in_specs=[pl.BlockSpec((RB, K), lambda i: (i, 0)),
                  pl.BlockSpec((K, K), lambda i: (0, 0))],
        out_specs=pl.BlockSpec((RB, K), lambda i: (i, 0)),
        out_shape=jax.ShapeDtypeStruct((n, K), jnp.float32),
    )(x, W1)

    g1, dinv = pl.pallas_call(
        _scale_body,
        grid=(G,),
        in_specs=[pl.BlockSpec((RB, K), lambda i: (i, 0)),
                  pl.BlockSpec((NC, RB, K), lambda i: (0, i, 0))],
        out_specs=[pl.BlockSpec((RB, K), lambda i: (i, 0)),
                   pl.BlockSpec((RB, 1), lambda i: (i, 0))],
        out_shape=[jax.ShapeDtypeStruct((n, K), jnp.float32),
                   jax.ShapeDtypeStruct((n, 1), jnp.float32)],
    )(h1, deg_parts.reshape(NC, NP, K)[:, :n, :])

    mp = _make_mp_kernel(EPW, K, NP)
    acc1 = mp(g1, src, dst).reshape(NC, NP, K)[:, :n, :]

    g2 = pl.pallas_call(
        _mid_body,
        grid=(G,),
        in_specs=[pl.BlockSpec((NC, RB, K), lambda i: (0, i, 0)),
                  pl.BlockSpec((RB, K), lambda i: (i, 0)),
                  pl.BlockSpec((RB, 1), lambda i: (i, 0)),
                  pl.BlockSpec((1, K), lambda i: (0, 0)),
                  pl.BlockSpec((K, K), lambda i: (0, 0))],
        out_specs=pl.BlockSpec((RB, K), lambda i: (i, 0)),
        out_shape=jax.ShapeDtypeStruct((n, K), jnp.float32),
    )(acc1, g1, dinv, b1.reshape(1, K), W2)

    acc2 = mp(g2, src, dst).reshape(NC, NP, K)[:, :n, :]

    out = pl.pallas_call(
        _out_body,
        grid=(G,),
        in_specs=[pl.BlockSpec((NC, RB, K), lambda i: (0, i, 0)),
                  pl.BlockSpec((RB, K), lambda i: (i, 0)),
                  pl.BlockSpec((RB, 1), lambda i: (i, 0)),
                  pl.BlockSpec((1, K), lambda i: (0, 0))],
        out_specs=pl.BlockSpec((RB, K), lambda i: (i, 0)),
        out_shape=jax.ShapeDtypeStruct((n, K), jnp.float32),
    )(acc2, g2, dinv, b2.reshape(1, K))

    return out


# speedup vs baseline: 2.8091x; 2.1701x over previous
import functools

import jax
import jax.numpy as jnp
from jax import lax
from jax.experimental import pallas as pl
from jax.experimental.pallas import tpu as pltpu
from jax.experimental.pallas import tpu_sc as plsc

NC = 2
NS = 16
L = 16


def _sc_mesh():
    return plsc.VectorSubcoreMesh(
        core_axis_name="c", subcore_axis_name="s", num_cores=NC, num_subcores=NS
    )


C = 40
NB = 5
NR = 5
ZR = 32


def _make_deg_kernel(EPW, NP, W):
    NCH = EPW // C
    RPT = NP // NS
    assert EPW % (C * NR) == 0 and NP % NS == 0 and RPT % ZR == 0 and W % L == 0

    @functools.partial(
        pl.kernel,
        out_type=jax.ShapeDtypeStruct((NC * NP, W), jnp.float32),
        mesh=_sc_mesh(),
        scratch_types=(
            [pltpu.VMEM((C,), jnp.int32)] * NR
            + [
                pltpu.VMEM((C, W), jnp.float32),
                pltpu.VMEM((ZR, W), jnp.float32),
                pltpu.VMEM_SHARED((NP, W), jnp.float32),
            ]
            + [pltpu.SemaphoreType.DMA] * NR
        ),
    )
    def deg_kernel(dst_hbm, out_hbm, *scr):
        dsti = scr[:NR]
        ones_v, zb_v, acc_sh = scr[NR:NR + 3]
        dsem = scr[NR + 3:]
        c = lax.axis_index("c")
        s = lax.axis_index("s")
        wid = c * NS + s
        base = wid * EPW
        one_row = jnp.where(lax.iota(jnp.int32, L) == 0, 1.0, 0.0)
        zero_row = jnp.zeros((L,), jnp.float32)

        for m in range(NR):
            pltpu.async_copy(dst_hbm.at[pl.ds(base + m * C, C)], dsti[m], dsem[m])

        @pl.loop(0, C)
        def _(i):
            ones_v[i, pl.ds(0, L)] = one_row

            @pl.loop(L, W, step=L)
            def _(j):
                ones_v[i, pl.ds(j, L)] = zero_row

        @pl.loop(0, ZR)
        def _(i):
            @pl.loop(0, W, step=L)
            def _(j):
                zb_v[i, pl.ds(j, L)] = zero_row

        @pl.loop(0, RPT, step=ZR)
        def _(r):
            pltpu.sync_copy(zb_v, acc_sh.at[pl.ds(s * RPT + r, ZR)])

        plsc.subcore_barrier()

        @pl.loop(0, NCH, step=NR)
        def _(g):
            for j in range(NR):
                k = g + j
                pltpu.make_async_copy(
                    dst_hbm.at[pl.ds(base + k * C, C)], dsti[j], dsem[j]
                ).wait()
                pltpu.sync_copy(ones_v, acc_sh.at[dsti[j]], add=True)
                nk = k + NR

                @pl.when(nk < NCH)
                def _():
                    pltpu.async_copy(
                        dst_hbm.at[pl.ds(base + nk * C, C)], dsti[j], dsem[j]
                    )

        plsc.subcore_barrier()
        pltpu.sync_copy(
            acc_sh.at[pl.ds(s * RPT, RPT)],
            out_hbm.at[pl.ds(c * NP + s * RPT, RPT)],
        )

    return deg_kernel


def _make_mp_kernel(EPW, D, NP):
    NCH = EPW // C
    RPT = NP // NS
    A = NB - 1
    assert EPW % (C * NB) == 0 and D % L == 0 and NCH >= 2 * NB
    assert NP % NS == 0 and RPT % ZR == 0

    @functools.partial(
        pl.kernel,
        out_type=jax.ShapeDtypeStruct((NC * NP, D), jnp.float32),
        mesh=_sc_mesh(),
        scratch_types=(
            [pltpu.VMEM((EPW,), jnp.int32)]
            + [pltpu.VMEM((C,), jnp.int32)] * NB
            + [pltpu.VMEM((C, D), jnp.float32)] * NB
            + [
                pltpu.VMEM((ZR, D), jnp.float32),
                pltpu.VMEM_SHARED((NP, D), jnp.float32),
            ]
            + [pltpu.SemaphoreType.DMA] * (3 * NB)
        ),
    )
    def mp_kernel(g_hbm, src_hbm, dst_hbm, out_hbm, *scr):
        srci = scr[0]
        dsti = scr[1:1 + NB]
        rows = scr[1 + NB:1 + 2 * NB]
        zb_v, acc_sh = scr[1 + 2 * NB:3 + 2 * NB]
        gsem = scr[3 + 2 * NB:3 + 3 * NB]
        dsem = scr[3 + 3 * NB:3 + 4 * NB]
        ssem = scr[3 + 4 * NB:3 + 5 * NB]
        c = lax.axis_index("c")
        s = lax.axis_index("s")
        wid = c * NS + s
        base = wid * EPW
        zero_row = jnp.zeros((L,), jnp.float32)

        for m in range(A):
            pltpu.async_copy(dst_hbm.at[pl.ds(base + m * C, C)], dsti[m], dsem[m])
        pltpu.sync_copy(src_hbm.at[pl.ds(base, EPW)], srci)

        @pl.loop(0, ZR)
        def _(i):
            @pl.loop(0, D, step=L)
            def _(j):
                zb_v[i, pl.ds(j, L)] = zero_row

        @pl.loop(0, RPT, step=ZR)
        def _(r):
            pltpu.sync_copy(zb_v, acc_sh.at[pl.ds(s * RPT + r, ZR)])

        plsc.subcore_barrier()

        for b in range(A):
            pltpu.async_copy(
                g_hbm.at[srci.at[pl.ds(b * C, C)]], rows[b], gsem[b]
            )

        @pl.loop(0, NCH, step=NB)
        def _(g):
            for j in range(NB):
                k = g + j
                pltpu.make_async_copy(
                    g_hbm.at[srci.at[pl.ds(k * C, C)]], rows[j], gsem[j]
                ).wait()
                pltpu.make_async_copy(
                    dst_hbm.at[pl.ds(base + k * C, C)], dsti[j], dsem[j]
                ).wait()
                pltpu.async_copy(rows[j], acc_sh.at[dsti[j]], ssem[j], add=True)
                nk = k + A
                bn = (j + A) % NB

                def _refill():
                    pltpu.async_copy(
                        dst_hbm.at[pl.ds(base + nk * C, C)], dsti[bn], dsem[bn]
                    )
                    pltpu.async_copy(
                        g_hbm.at[srci.at[pl.ds(nk * C, C)]], rows[bn], gsem[bn]
                    )

                if j == 0:
                    @pl.when(g == 0)
                    def _():
                        _refill()

                    @pl.when((g > 0) & (nk < NCH))
                    def _():
                        pltpu.make_async_copy(
                            rows[bn], acc_sh.at[dsti[bn]], ssem[bn]
                        ).wait()
                        _refill()
                else:
                    @pl.when(nk < NCH)
                    def _():
                        pltpu.make_async_copy(
                            rows[bn], acc_sh.at[dsti[bn]], ssem[bn]
                        ).wait()
                        _refill()

        for b in range(NB):
            pltpu.make_async_copy(rows[b], acc_sh.at[dsti[b]], ssem[b]).wait()

        plsc.subcore_barrier()
        pltpu.sync_copy(
            acc_sh.at[pl.ds(s * RPT, RPT)],
            out_hbm.at[pl.ds(c * NP + s * RPT, RPT)],
        )

    return mp_kernel


def _mm_body(x_ref, w_ref, o_ref):
    o_ref[...] = jnp.dot(x_ref[...], w_ref[...],
                         preferred_element_type=jnp.float32)


def _scale_body(h_ref, dp_ref, g_ref, dinv_ref):
    p = dp_ref[...]
    deg = jnp.sum(p[0] + p[1], axis=1, keepdims=True) + 1.0
    dinv = lax.rsqrt(deg)
    dinv_ref[...] = dinv
    g_ref[...] = dinv * h_ref[...]


def _mid_body(a_ref, g_ref, dinv_ref, b_ref, w_ref, o_ref):
    dinv = dinv_ref[...]
    acc = a_ref[0] + a_ref[1] + g_ref[...]
    z = jnp.maximum(dinv * acc + b_ref[...], 0.0)
    o_ref[...] = dinv * jnp.dot(z, w_ref[...],
                                preferred_element_type=jnp.float32)


def _out_body(a_ref, g_ref, dinv_ref, b_ref, o_ref):
    o_ref[...] = dinv_ref[...] * (a_ref[0] + a_ref[1] + g_ref[...]) + b_ref[...]


def kernel(x, edge_index, W1, b1, W2, b2):
    n, K = x.shape
    E = edge_index.shape[1]
    RB = 1000
    assert n % RB == 0
    G = n // RB
    NP = -(-n // 2048) * 2048
    NW = NC * NS
    EPW = -(-E // (NW * C * NR)) * C * NR
    PAD = EPW * NW - E
    if PAD > 0 and NP == n:
        NP += 2048
    src = edge_index[0]
    dst = edge_index[1]
    if PAD > 0:
        ppw = PAD // NW
        dump = n + (jnp.arange(PAD, dtype=jnp.int32) % (NP - n))
        if E % NW == 0 and PAD % NW == 0:
            src = jnp.concatenate(
                [src.reshape(NW, -1), jnp.zeros((NW, ppw), jnp.int32)], axis=1
            ).reshape(-1)
            dst = jnp.concatenate(
                [dst.reshape(NW, -1), dump.reshape(NW, ppw)], axis=1
            ).reshape(-1)
        else:
            src = jnp.concatenate([src, jnp.zeros((PAD,), jnp.int32)])
            dst = jnp.concatenate([dst, dump])

    deg_parts = _make_deg_kernel(EPW, NP, K)(dst)

    h1 = pl.pallas_call(
        _mm_body,
        grid=(G,),
        in_specs=[pl.BlockSpec((RB, K), lambda i: (i, 0)),
                  pl.BlockSpec((K, K), lambda i: (0, 0))],
        out_specs=pl.BlockSpec((RB, K), lambda i: (i, 0)),
        out_shape=jax.ShapeDtypeStruct((n, K), jnp.float32),
    )(x, W1)

    g1, dinv = pl.pallas_call(
        _scale_body,
        grid=(G,),
        in_specs=[pl.BlockSpec((RB, K), lambda i: (i, 0)),
                  pl.BlockSpec((NC, RB, K), lambda i: (0, i, 0))],
        out_specs=[pl.BlockSpec((RB, K), lambda i: (i, 0)),
                   pl.BlockSpec((RB, 1), lambda i: (i, 0))],
        out_shape=[jax.ShapeDtypeStruct((n, K), jnp.float32),
                   jax.ShapeDtypeStruct((n, 1), jnp.float32)],
    )(h1, deg_parts.reshape(NC, NP, K)[:, :n, :])

    mp = _make_mp_kernel(EPW, K, NP)
    acc1 = mp(g1, src, dst).reshape(NC, NP, K)[:, :n, :]

    g2 = pl.pallas_call(
        _mid_body,
        grid=(G,),
        in_specs=[pl.BlockSpec((NC, RB, K), lambda i: (0, i, 0)),
                  pl.BlockSpec((RB, K), lambda i: (i, 0)),
                  pl.BlockSpec((RB, 1), lambda i: (i, 0)),
                  pl.BlockSpec((1, K), lambda i: (0, 0)),
                  pl.BlockSpec((K, K), lambda i: (0, 0))],
        out_specs=pl.BlockSpec((RB, K), lambda i: (i, 0)),
        out_shape=jax.ShapeDtypeStruct((n, K), jnp.float32),
    )(acc1, g1, dinv, b1.reshape(1, K), W2)

    acc2 = mp(g2, src, dst).reshape(NC, NP, K)[:, :n, :]

    out = pl.pallas_call(
        _out_body,
        grid=(G,),
        in_specs=[pl.BlockSpec((NC, RB, K), lambda i: (0, i, 0)),
                  pl.BlockSpec((RB, K), lambda i: (i, 0)),
                  pl.BlockSpec((RB, 1), lambda i: (i, 0)),
                  pl.BlockSpec((1, K), lambda i: (0, 0))],
        out_specs=pl.BlockSpec((RB, K), lambda i: (i, 0)),
        out_shape=jax.ShapeDtypeStruct((n, K), jnp.float32),
    )(acc2, g2, dinv, b2.reshape(1, K))

    return out


# speedup vs baseline: 2.8923x; 1.0296x over previous
import functools

import jax
import jax.numpy as jnp
from jax import lax
from jax.experimental import pallas as pl
from jax.experimental.pallas import tpu as pltpu
from jax.experimental.pallas import tpu_sc as plsc

NC = 2
NS = 16
L = 16


def _sc_mesh():
    return plsc.VectorSubcoreMesh(
        core_axis_name="c", subcore_axis_name="s", num_cores=NC, num_subcores=NS
    )


C = 40
NB = 5
NR = 5
ZR = 32


def _make_deg_kernel(EPW, NP, W):
    NCH = EPW // C
    RPT = NP // NS
    A = NR - 1
    assert EPW % (C * NR) == 0 and NP % NS == 0 and RPT % ZR == 0 and W % L == 0
    assert NCH >= 2 * NR

    @functools.partial(
        pl.kernel,
        out_type=jax.ShapeDtypeStruct((NC * NP, W), jnp.float32),
        mesh=_sc_mesh(),
        scratch_types=(
            [pltpu.VMEM((C,), jnp.int32)] * NR
            + [
                pltpu.VMEM((C, W), jnp.float32),
                pltpu.VMEM((ZR, W), jnp.float32),
                pltpu.VMEM_SHARED((NP, W), jnp.float32),
            ]
            + [pltpu.SemaphoreType.DMA] * (2 * NR)
        ),
    )
    def deg_kernel(dst_hbm, out_hbm, *scr):
        dsti = scr[:NR]
        ones_v, zb_v, acc_sh = scr[NR:NR + 3]
        dsem = scr[NR + 3:NR + 3 + NR]
        ssem = scr[NR + 3 + NR:]
        c = lax.axis_index("c")
        s = lax.axis_index("s")
        wid = c * NS + s
        base = wid * EPW
        one_row = jnp.where(lax.iota(jnp.int32, L) == 0, 1.0, 0.0)
        zero_row = jnp.zeros((L,), jnp.float32)

        for m in range(A):
            pltpu.async_copy(dst_hbm.at[pl.ds(base + m * C, C)], dsti[m], dsem[m])

        @pl.loop(0, C)
        def _(i):
            ones_v[i, pl.ds(0, L)] = one_row

            @pl.loop(L, W, step=L)
            def _(j):
                ones_v[i, pl.ds(j, L)] = zero_row

        @pl.loop(0, ZR)
        def _(i):
            @pl.loop(0, W, step=L)
            def _(j):
                zb_v[i, pl.ds(j, L)] = zero_row

        @pl.loop(0, RPT, step=ZR)
        def _(r):
            pltpu.sync_copy(zb_v, acc_sh.at[pl.ds(s * RPT + r, ZR)])

        plsc.subcore_barrier()

        @pl.loop(0, NCH, step=NR)
        def _(g):
            for j in range(NR):
                k = g + j
                pltpu.make_async_copy(
                    dst_hbm.at[pl.ds(base + k * C, C)], dsti[j], dsem[j]
                ).wait()
                pltpu.async_copy(ones_v, acc_sh.at[dsti[j]], ssem[j], add=True)
                nk = k + A
                bn = (j + A) % NR

                def _refill():
                    pltpu.async_copy(
                        dst_hbm.at[pl.ds(base + nk * C, C)], dsti[bn], dsem[bn]
                    )

                if j == 0:
                    @pl.when(g == 0)
                    def _():
                        _refill()

                    @pl.when((g > 0) & (nk < NCH))
                    def _():
                        pltpu.make_async_copy(
                            ones_v, acc_sh.at[dsti[bn]], ssem[bn]
                        ).wait()
                        _refill()
                else:
                    @pl.when(nk < NCH)
                    def _():
                        pltpu.make_async_copy(
                            ones_v, acc_sh.at[dsti[bn]], ssem[bn]
                        ).wait()
                        _refill()

        for b in range(NR):
            pltpu.make_async_copy(ones_v, acc_sh.at[dsti[b]], ssem[b]).wait()

        plsc.subcore_barrier()
        pltpu.sync_copy(
            acc_sh.at[pl.ds(s * RPT, RPT)],
            out_hbm.at[pl.ds(c * NP + s * RPT, RPT)],
        )

    return deg_kernel


def _make_mp_kernel(EPW, D, NP):
    NCH = EPW // C
    RPT = NP // NS
    A = NB - 1
    assert EPW % (C * NB) == 0 and D % L == 0 and NCH >= 2 * NB
    assert NP % NS == 0 and RPT % ZR == 0

    @functools.partial(
        pl.kernel,
        out_type=jax.ShapeDtypeStruct((NC * NP, D), jnp.float32),
        mesh=_sc_mesh(),
        scratch_types=(
            [pltpu.VMEM((EPW,), jnp.int32)]
            + [pltpu.VMEM((C,), jnp.int32)] * NB
            + [pltpu.VMEM((C, D), jnp.float32)] * NB
            + [
                pltpu.VMEM((ZR, D), jnp.float32),
                pltpu.VMEM_SHARED((NP, D), jnp.float32),
            ]
            + [pltpu.SemaphoreType.DMA] * (3 * NB)
        ),
    )
    def mp_kernel(g_hbm, src_hbm, dst_hbm, out_hbm, *scr):
        srci = scr[0]
        dsti = scr[1:1 + NB]
        rows = scr[1 + NB:1 + 2 * NB]
        zb_v, acc_sh = scr[1 + 2 * NB:3 + 2 * NB]
        gsem = scr[3 + 2 * NB:3 + 3 * NB]
        dsem = scr[3 + 3 * NB:3 + 4 * NB]
        ssem = scr[3 + 4 * NB:3 + 5 * NB]
        c = lax.axis_index("c")
        s = lax.axis_index("s")
        wid = c * NS + s
        base = wid * EPW
        zero_row = jnp.zeros((L,), jnp.float32)

        for m in range(A):
            pltpu.async_copy(dst_hbm.at[pl.ds(base + m * C, C)], dsti[m], dsem[m])
        pltpu.sync_copy(src_hbm.at[pl.ds(base, EPW)], srci)

        @pl.loop(0, ZR)
        def _(i):
            @pl.loop(0, D, step=L)
            def _(j):
                zb_v[i, pl.ds(j, L)] = zero_row

        @pl.loop(0, RPT, step=ZR)
        def _(r):
            pltpu.sync_copy(zb_v, acc_sh.at[pl.ds(s * RPT + r, ZR)])

        plsc.subcore_barrier()

        for b in range(A):
            pltpu.async_copy(
                g_hbm.at[srci.at[pl.ds(b * C, C)]], rows[b], gsem[b]
            )

        @pl.loop(0, NCH, step=NB)
        def _(g):
            for j in range(NB):
                k = g + j
                pltpu.make_async_copy(
                    g_hbm.at[srci.at[pl.ds(k * C, C)]], rows[j], gsem[j]
                ).wait()
                pltpu.make_async_copy(
                    dst_hbm.at[pl.ds(base + k * C, C)], dsti[j], dsem[j]
                ).wait()
                pltpu.async_copy(rows[j], acc_sh.at[dsti[j]], ssem[j], add=True)
                nk = k + A
                bn = (j + A) % NB

                def _refill():
                    pltpu.async_copy(
                        dst_hbm.at[pl.ds(base + nk * C, C)], dsti[bn], dsem[bn]
                    )
                    pltpu.async_copy(
                        g_hbm.at[srci.at[pl.ds(nk * C, C)]], rows[bn], gsem[bn]
                    )

                if j == 0:
                    @pl.when(g == 0)
                    def _():
                        _refill()

                    @pl.when((g > 0) & (nk < NCH))
                    def _():
                        pltpu.make_async_copy(
                            rows[bn], acc_sh.at[dsti[bn]], ssem[bn]
                        ).wait()
                        _refill()
                else:
                    @pl.when(nk < NCH)
                    def _():
                        pltpu.make_async_copy(
                            rows[bn], acc_sh.at[dsti[bn]], ssem[bn]
                        ).wait()
                        _refill()

        for b in range(NB):
            pltpu.make_async_copy(rows[b], acc_sh.at[dsti[b]], ssem[b]).wait()

        plsc.subcore_barrier()
        pltpu.sync_copy(
            acc_sh.at[pl.ds(s * RPT, RPT)],
            out_hbm.at[pl.ds(c * NP + s * RPT, RPT)],
        )

    return mp_kernel


def _mm_body(x_ref, w_ref, o_ref):
    o_ref[...] = jnp.dot(x_ref[...], w_ref[...],
                         preferred_element_type=jnp.float32)


def _scale_body(h_ref, dp_ref, g_ref, dinv_ref):
    p = dp_ref[...]
    deg = jnp.sum(p[0] + p[1], axis=1, keepdims=True) + 1.0
    dinv = lax.rsqrt(deg)
    dinv_ref[...] = dinv
    g_ref[...] = dinv * h_ref[...]


def _mid_body(a_ref, g_ref, dinv_ref, b_ref, w_ref, o_ref):
    dinv = dinv_ref[...]
    acc = a_ref[0] + a_ref[1] + g_ref[...]
    z = jnp.maximum(dinv * acc + b_ref[...], 0.0)
    o_ref[...] = dinv * jnp.dot(z, w_ref[...],
                                preferred_element_type=jnp.float32)


def _out_body(a_ref, g_ref, dinv_ref, b_ref, o_ref):
    o_ref[...] = dinv_ref[...] * (a_ref[0] + a_ref[1] + g_ref[...]) + b_ref[...]


def kernel(x, edge_index, W1, b1, W2, b2):
    n, K = x.shape
    E = edge_index.shape[1]
    RB = 1000
    assert n % RB == 0
    G = n // RB
    NP = -(-n // 2048) * 2048
    NW = NC * NS
    EPW = -(-E // (NW * C * NR)) * C * NR
    PAD = EPW * NW - E
    if PAD > 0 and NP == n:
        NP += 2048
    src = edge_index[0]
    dst = edge_index[1]
    if PAD > 0:
        ppw = PAD // NW
        dump = n + (jnp.arange(PAD, dtype=jnp.int32) % (NP - n))
        if E % NW == 0 and PAD % NW == 0:
            src = jnp.concatenate(
                [src.reshape(NW, -1), jnp.zeros((NW, ppw), jnp.int32)], axis=1
            ).reshape(-1)
            dst = jnp.concatenate(
                [dst.reshape(NW, -1), dump.reshape(NW, ppw)], axis=1
            ).reshape(-1)
        else:
            src = jnp.concatenate([src, jnp.zeros((PAD,), jnp.int32)])
            dst = jnp.concatenate([dst, dump])

    deg_parts = _make_deg_kernel(EPW, NP, K)(dst)

    h1 = pl.pallas_call(
        _mm_body,
        grid=(G,),
        in_specs=[pl.BlockSpec((RB, K), lambda i: (i, 0)),
                  pl.BlockSpec((K, K), lambda i: (0, 0))],
        out_specs=pl.BlockSpec((RB, K), lambda i: (i, 0)),
        out_shape=jax.ShapeDtypeStruct((n, K), jnp.float32),
    )(x, W1)

    g1, dinv = pl.pallas_call(
        _scale_body,
        grid=(G,),
        in_specs=[pl.BlockSpec((RB, K), lambda i: (i, 0)),
                  pl.BlockSpec((NC, RB, K), lambda i: (0, i, 0))],
        out_specs=[pl.BlockSpec((RB, K), lambda i: (i, 0)),
                   pl.BlockSpec((RB, 1), lambda i: (i, 0))],
        out_shape=[jax.ShapeDtypeStruct((n, K), jnp.float32),
                   jax.ShapeDtypeStruct((n, 1), jnp.float32)],
    )(h1, deg_parts.reshape(NC, NP, K)[:, :n, :])

    mp = _make_mp_kernel(EPW, K, NP)
    acc1 = mp(g1, src, dst).reshape(NC, NP, K)[:, :n, :]

    g2 = pl.pallas_call(
        _mid_body,
        grid=(G,),
        in_specs=[pl.BlockSpec((NC, RB, K), lambda i: (0, i, 0)),
                  pl.BlockSpec((RB, K), lambda i: (i, 0)),
                  pl.BlockSpec((RB, 1), lambda i: (i, 0)),
                  pl.BlockSpec((1, K), lambda i: (0, 0)),
                  pl.BlockSpec((K, K), lambda i: (0, 0))],
        out_specs=pl.BlockSpec((RB, K), lambda i: (i, 0)),
        out_shape=jax.ShapeDtypeStruct((n, K), jnp.float32),
    )(acc1, g1, dinv, b1.reshape(1, K), W2)

    acc2 = mp(g2, src, dst).reshape(NC, NP, K)[:, :n, :]

    out = pl.pallas_call(
        _out_body,
        grid=(G,),
        in_specs=[pl.BlockSpec((NC, RB, K), lambda i: (0, i, 0)),
                  pl.BlockSpec((RB, K), lambda i: (i, 0)),
                  pl.BlockSpec((RB, 1), lambda i: (i, 0)),
                  pl.BlockSpec((1, K), lambda i: (0, 0))],
        out_specs=pl.BlockSpec((RB, K), lambda i: (i, 0)),
        out_shape=jax.ShapeDtypeStruct((n, K), jnp.float32),
    )(acc2, g2, dinv, b2.reshape(1, K))

    return out
